# Initial kernel scaffold; baseline (speedup 1.0000x reference)
import jax
import jax.numpy as jnp
from jax.experimental import pallas as pl


def kernel(theta, x, edge_index, Wl1, bl1, Wr1, Wl2, bl2, Wr2, Wfc, bfc, Wn1, bn1, Wn2, bn2, Wn3, bn3, Wn4, bn4, Wno, bno):
    raise NotImplementedError("write your pallas kernel here")


# speedup vs baseline: 13.7123x; 13.7123x over previous
import functools

import jax
import jax.numpy as jnp
from jax import lax
from jax.experimental import pallas as pl
from jax.experimental.pallas import tpu as pltpu
from jax.experimental.pallas import tpu_sc as plsc

N = 100000
E = 3200000
F = 16
B = 1024
TD = 8
H = 128

NC = 2
NS = 16
NW = NC * NS
NSLICE = 6272
NP = NS * NSLICE
EP_ROWS = 25088
EP = EP_ROWS * 128
CK = 16
RP1 = EP_ROWS // NW
RP2 = EP_ROWS // NS

BN = 2000
GRID_N = N // BN


def _elu(v):
    return jnp.where(v > 0, v, jnp.exp(v) - 1.0)


def _sc_mesh():
    return plsc.VectorSubcoreMesh(core_axis_name="c", subcore_axis_name="s",
                                  num_cores=NC, num_subcores=NS)


_SC_PARAMS = pltpu.CompilerParams(use_tc_tiling_on_sc=False)


def _sc_pass1(xt, src2, dst2, zagg, zcnt):
    mesh = _sc_mesh()

    @functools.partial(
        pl.kernel,
        out_type=[
            jax.ShapeDtypeStruct((NC, NP, F), jnp.float32),
            jax.ShapeDtypeStruct((NC, NP), jnp.float32),
        ],
        mesh=mesh,
        compiler_params=_SC_PARAMS,
        scratch_types=[
            pltpu.VMEM((CK, 128), jnp.int32),
            pltpu.VMEM((CK, 128), jnp.int32),
            pltpu.VMEM((128, F), jnp.float32),
            pltpu.VMEM((128,), jnp.float32),
            pltpu.VMEM_SHARED((NP, F), jnp.float32),
            pltpu.VMEM_SHARED((NP,), jnp.float32),
        ],
    )
    def k(xt_hbm, src_hbm, dst_hbm, zagg_hbm, zcnt_hbm, aggp_hbm, cntp_hbm,
          sidx, didx, rows, ones, agg_sh, cnt_sh):
        cid = lax.axis_index("c")
        sid = lax.axis_index("s")
        wid = cid * NS + sid
        for t in range(128 // 16):
            ones[pl.ds(t * 16, 16)] = jnp.ones((16,), jnp.float32)
        sl = pl.ds(sid * NSLICE, NSLICE)
        pltpu.sync_copy(zagg_hbm, agg_sh.at[sl])
        pltpu.sync_copy(zcnt_hbm, cnt_sh.at[sl])
        plsc.subcore_barrier()

        base = wid * RP1

        @pl.loop(0, RP1 // CK)
        def _(i):
            r0 = base + i * CK
            pltpu.sync_copy(src_hbm.at[pl.ds(r0, CK)], sidx)
            pltpu.sync_copy(dst_hbm.at[pl.ds(r0, CK)], didx)
            for j in range(CK):
                pltpu.sync_copy(xt_hbm.at[sidx.at[j]], rows)
                pltpu.sync_copy(rows, agg_sh.at[didx.at[j]], add=True)
                pltpu.sync_copy(ones, cnt_sh.at[didx.at[j]], add=True)

        plsc.subcore_barrier()
        pltpu.sync_copy(agg_sh.at[sl], aggp_hbm.at[cid].at[sl])
        pltpu.sync_copy(cnt_sh.at[sl], cntp_hbm.at[cid].at[sl])

    return k(xt, src2, dst2, zagg, zcnt)


def _sc_pass2(ta, tb, src2, dst2, zagg):
    mesh = _sc_mesh()

    @functools.partial(
        pl.kernel,
        out_type=jax.ShapeDtypeStruct((NC, NP, F), jnp.float32),
        mesh=mesh,
        compiler_params=_SC_PARAMS,
        scratch_types=[
            pltpu.VMEM((CK, 128), jnp.int32),
            pltpu.VMEM((CK, 128), jnp.int32),
            pltpu.VMEM((128, F), jnp.float32),
            pltpu.VMEM_SHARED((NP, F), jnp.float32),
        ],
    )
    def k(ta_hbm, tb_hbm, src_hbm, dst_hbm, zagg_hbm, aggp_hbm,
          sidx, didx, rows, agg_sh):
        cid = lax.axis_index("c")
        sid = lax.axis_index("s")
        sl = pl.ds(sid * NSLICE, NSLICE)
        pltpu.sync_copy(zagg_hbm, agg_sh.at[sl])
        plsc.subcore_barrier()

        def edge_loop(tab_hbm):
            base = sid * RP2

            @pl.loop(0, RP2 // CK)
            def _(i):
                r0 = base + i * CK
                pltpu.sync_copy(src_hbm.at[pl.ds(r0, CK)], sidx)
                pltpu.sync_copy(dst_hbm.at[pl.ds(r0, CK)], didx)
                for j in range(CK):
                    pltpu.sync_copy(tab_hbm.at[sidx.at[j]], rows)
                    pltpu.sync_copy(rows, agg_sh.at[didx.at[j]], add=True)

        @pl.when(cid == 0)
        def _():
            edge_loop(ta_hbm)

        @pl.when(cid == 1)
        def _():
            edge_loop(tb_hbm)

        plsc.subcore_barrier()
        pltpu.sync_copy(agg_sh.at[sl], aggp_hbm.at[cid].at[sl])

    return k(ta, tb, src2, dst2, zagg)


def _dense1(aggp, cntp3, x, Wl1, bl1, Wr1, Wl2, Wr2):
    def body(aggp_ref, cntp_ref, x_ref, wl1_ref, bl1_ref, wr1_ref,
             wl2_ref, wr2_ref, p1a_ref, p1b_ref, q_ref, cnti_ref):
        agg = aggp_ref[0] + aggp_ref[1]
        cnt = cntp_ref[0] + cntp_ref[1]
        cnti = 1.0 / jnp.maximum(cnt, 1.0)
        mean = agg * cnti
        dn = (((1,), (1,)), ((), ()))
        out1 = (lax.dot_general(mean, wl1_ref[...], dn,
                                preferred_element_type=jnp.float32)
                + bl1_ref[...]
                + lax.dot_general(x_ref[...], wr1_ref[...], dn,
                                  preferred_element_type=jnp.float32))
        rn = jnp.sqrt(jnp.sum(out1 * out1, axis=1, keepdims=True))
        out1 = out1 / jnp.maximum(rn, 1e-12)
        h1 = _elu(out1)
        p1 = lax.dot_general(h1, wl2_ref[...], dn,
                             preferred_element_type=jnp.float32)
        q = lax.dot_general(h1, wr2_ref[...], dn,
                            preferred_element_type=jnp.float32)
        p1a_ref[...] = p1[:, :F]
        p1b_ref[...] = p1[:, F:]
        q_ref[...] = q
        cnti_ref[...] = cnti

    full = lambda shape: pl.BlockSpec(shape, lambda i: tuple(0 for _ in shape))
    return pl.pallas_call(
        body,
        grid=(GRID_N,),
        in_specs=[
            pl.BlockSpec((NC, BN, F), lambda i: (0, i, 0)),
            pl.BlockSpec((NC, BN, 1), lambda i: (0, i, 0)),
            pl.BlockSpec((BN, F), lambda i: (i, 0)),
            full((4 * F, F)),
            full((1, 4 * F)),
            full((4 * F, F)),
            full((2 * F, 4 * F)),
            full((2 * F, 4 * F)),
        ],
        out_specs=[
            pl.BlockSpec((BN, F), lambda i: (i, 0)),
            pl.BlockSpec((BN, F), lambda i: (i, 0)),
            pl.BlockSpec((BN, 2 * F), lambda i: (i, 0)),
            pl.BlockSpec((BN, 1), lambda i: (i, 0)),
        ],
        out_shape=[
            jax.ShapeDtypeStruct((N, F), jnp.float32),
            jax.ShapeDtypeStruct((N, F), jnp.float32),
            jax.ShapeDtypeStruct((N, 2 * F), jnp.float32),
            jax.ShapeDtypeStruct((N, 1), jnp.float32),
        ],
    )(aggp, cntp3, x, Wl1, bl1.reshape(1, 4 * F), Wr1, Wl2, Wr2)


def _dense2(agg2p, cnti, q, bl2):
    def body(a2_ref, cnti_ref, q_ref, bl2_ref, gm_ref):
        agg2 = jnp.concatenate([a2_ref[0], a2_ref[1]], axis=1)
        out2 = agg2 * cnti_ref[...] + bl2_ref[...] + q_ref[...]
        rn = jnp.sqrt(jnp.sum(out2 * out2, axis=1, keepdims=True))
        out2 = out2 / jnp.maximum(rn, 1e-12)
        h2 = _elu(out2)
        m = jnp.max(h2, axis=0, keepdims=True)

        @pl.when(pl.program_id(0) == 0)
        def _():
            gm_ref[...] = jnp.full((1, 2 * F), -jnp.inf, jnp.float32)

        gm_ref[...] = jnp.maximum(gm_ref[...], m)

    return pl.pallas_call(
        body,
        grid=(GRID_N,),
        in_specs=[
            pl.BlockSpec((NC, BN, F), lambda i: (0, i, 0)),
            pl.BlockSpec((BN, 1), lambda i: (i, 0)),
            pl.BlockSpec((BN, 2 * F), lambda i: (i, 0)),
            pl.BlockSpec((1, 2 * F), lambda i: (0, 0)),
        ],
        out_specs=pl.BlockSpec((1, 2 * F), lambda i: (0, 0)),
        out_shape=jax.ShapeDtypeStruct((1, 2 * F), jnp.float32),
    )(agg2p, cnti, q, bl2.reshape(1, 2 * F))


def _head(gmT, thetaT, Wfc, bfc, Wn1, bn1, Wn2, bn2, Wn3, bn3, Wn4, bn4,
          Wno, bno):
    def body(gm_ref, th_ref, wfc_ref, bfc_ref, w1_ref, b1_ref, w2_ref, b2_ref,
             w3_ref, b3_ref, w4_ref, b4_ref, wo_ref, bo_ref, out_ref):
        def mv(w_ref, v, b_ref):
            return jnp.dot(w_ref[...], v,
                           preferred_element_type=jnp.float32) + b_ref[...]

        g = _elu(gm_ref[...])
        g2 = mv(wfc_ref, g, bfc_ref)
        c = _elu(mv(w1_ref, g2, b1_ref))
        c = _elu(mv(w2_ref, c, b2_ref))
        c = _elu(mv(w3_ref, c, b3_ref))
        c = _elu(mv(w4_ref, c, b4_ref))
        st = mv(wo_ref, c, bo_ref)
        mu = st[:TD, :]
        ls = st[TD:, :]
        z = (th_ref[...] - mu) * jnp.exp(-ls)
        const = (jnp.sum(ls) + 0.5 * TD * jnp.log(2.0 * jnp.pi))
        out_ref[...] = -0.5 * jnp.sum(z * z, axis=0, keepdims=True) - const

    args = [gmT, thetaT, Wfc, bfc.reshape(F, 1), Wn1, bn1.reshape(H, 1),
            Wn2, bn2.reshape(H, 1), Wn3, bn3.reshape(H, 1),
            Wn4, bn4.reshape(H, 1), Wno, bno.reshape(2 * TD, 1)]
    return pl.pallas_call(
        body,
        out_shape=jax.ShapeDtypeStruct((1, B), jnp.float32),
    )(*args)


def kernel(theta, x, edge_index, Wl1, bl1, Wr1, Wl2, bl2, Wr2, Wfc, bfc,
           Wn1, bn1, Wn2, bn2, Wn3, bn3, Wn4, bn4, Wno, bno):
    src = edge_index[0].astype(jnp.int32)
    dst = edge_index[1].astype(jnp.int32)
    pad = EP - E
    srcp = jnp.concatenate([src, jnp.zeros((pad,), jnp.int32)])
    dstp = jnp.concatenate([dst, jnp.full((pad,), N, jnp.int32)])
    src2 = srcp.reshape(EP_ROWS, 128)
    dst2 = dstp.reshape(EP_ROWS, 128)
    zagg = jnp.zeros((NSLICE, F), jnp.float32)
    zcnt = jnp.zeros((NSLICE,), jnp.float32)

    aggp, cntp = _sc_pass1(x, src2, dst2, zagg, zcnt)
    p1a, p1b, q, cnti = _dense1(aggp, cntp.reshape(NC, NP, 1), x,
                                Wl1, bl1, Wr1, Wl2, Wr2)
    agg2p = _sc_pass2(p1a, p1b, src2, dst2, zagg)
    gm = _dense2(agg2p, cnti, q, bl2)
    logp = _head(gm.reshape(2 * F, 1), theta.T, Wfc, bfc, Wn1, bn1,
                 Wn2, bn2, Wn3, bn3, Wn4, bn4, Wno, bno)
    return logp[0]


# speedup vs baseline: 23.5026x; 1.7140x over previous
import functools

import jax
import jax.numpy as jnp
from jax import lax
from jax.experimental import pallas as pl
from jax.experimental.pallas import tpu as pltpu
from jax.experimental.pallas import tpu_sc as plsc

N = 100000
E = 3200000
F = 16
B = 1024
TD = 8
H = 128

NC = 2
NS = 16
NW = NC * NS
NSLICE = 6272
NP = NS * NSLICE
EP_ROWS = 25088
EP = EP_ROWS * 128
CE1 = 1024
CE2 = 1568
ET1 = EP // NW
ET2 = EP // NS

BN = 2000
GRID_N = N // BN


def _elu(v):
    return jnp.where(v > 0, v, jnp.exp(v) - 1.0)


def _sc_mesh():
    return plsc.VectorSubcoreMesh(core_axis_name="c", subcore_axis_name="s",
                                  num_cores=NC, num_subcores=NS)


_SC_PARAMS = pltpu.CompilerParams(use_tc_tiling_on_sc=False)


def _sc_pass1(xt, src2, dst2, zagg, zcnt):
    mesh = _sc_mesh()

    @functools.partial(
        pl.kernel,
        out_type=[
            jax.ShapeDtypeStruct((NC, NP, F), jnp.float32),
            jax.ShapeDtypeStruct((NC, NP), jnp.float32),
        ],
        mesh=mesh,
        compiler_params=_SC_PARAMS,
        scratch_types=[
            pltpu.VMEM((CE1,), jnp.int32),
            pltpu.VMEM((CE1,), jnp.int32),
            pltpu.VMEM((CE1, F), jnp.float32),
            pltpu.VMEM((CE1,), jnp.float32),
            pltpu.VMEM_SHARED((NP, F), jnp.float32),
            pltpu.VMEM_SHARED((NP,), jnp.float32),
        ],
    )
    def k(xt_hbm, src_hbm, dst_hbm, zagg_hbm, zcnt_hbm, aggp_hbm, cntp_hbm,
          sidx, didx, rows, ones, agg_sh, cnt_sh):
        cid = lax.axis_index("c")
        sid = lax.axis_index("s")
        wid = cid * NS + sid
        for t in range(CE1 // 16):
            ones[pl.ds(t * 16, 16)] = jnp.ones((16,), jnp.float32)
        sl = pl.ds(sid * NSLICE, NSLICE)
        pltpu.sync_copy(zagg_hbm, agg_sh.at[sl])
        pltpu.sync_copy(zcnt_hbm, cnt_sh.at[sl])
        plsc.subcore_barrier()

        base = wid * ET1

        @pl.loop(0, ET1 // CE1)
        def _(i):
            e0 = base + i * CE1
            pltpu.sync_copy(src_hbm.at[pl.ds(e0, CE1)], sidx)
            pltpu.sync_copy(dst_hbm.at[pl.ds(e0, CE1)], didx)
            pltpu.sync_copy(xt_hbm.at[sidx], rows)
            pltpu.sync_copy(rows, agg_sh.at[didx], add=True)
            pltpu.sync_copy(ones, cnt_sh.at[didx], add=True)

        plsc.subcore_barrier()
        pltpu.sync_copy(agg_sh.at[sl], aggp_hbm.at[cid].at[sl])
        pltpu.sync_copy(cnt_sh.at[sl], cntp_hbm.at[cid].at[sl])

    return k(xt, src2, dst2, zagg, zcnt)


def _sc_pass2(ta, tb, src2, dst2, zagg):
    mesh = _sc_mesh()

    @functools.partial(
        pl.kernel,
        out_type=jax.ShapeDtypeStruct((NC, NP, F), jnp.float32),
        mesh=mesh,
        compiler_params=_SC_PARAMS,
        scratch_types=[
            pltpu.VMEM((CE2,), jnp.int32),
            pltpu.VMEM((CE2,), jnp.int32),
            pltpu.VMEM((CE2, F), jnp.float32),
            pltpu.VMEM_SHARED((NP, F), jnp.float32),
        ],
    )
    def k(ta_hbm, tb_hbm, src_hbm, dst_hbm, zagg_hbm, aggp_hbm,
          sidx, didx, rows, agg_sh):
        cid = lax.axis_index("c")
        sid = lax.axis_index("s")
        sl = pl.ds(sid * NSLICE, NSLICE)
        pltpu.sync_copy(zagg_hbm, agg_sh.at[sl])
        plsc.subcore_barrier()

        def edge_loop(tab_hbm):
            base = sid * ET2

            @pl.loop(0, ET2 // CE2)
            def _(i):
                e0 = base + i * CE2
                pltpu.sync_copy(src_hbm.at[pl.ds(e0, CE2)], sidx)
                pltpu.sync_copy(dst_hbm.at[pl.ds(e0, CE2)], didx)
                pltpu.sync_copy(tab_hbm.at[sidx], rows)
                pltpu.sync_copy(rows, agg_sh.at[didx], add=True)

        @pl.when(cid == 0)
        def _():
            edge_loop(ta_hbm)

        @pl.when(cid == 1)
        def _():
            edge_loop(tb_hbm)

        plsc.subcore_barrier()
        pltpu.sync_copy(agg_sh.at[sl], aggp_hbm.at[cid].at[sl])

    return k(ta, tb, src2, dst2, zagg)


def _dense1(aggp, cntp3, x, Wl1, bl1, Wr1, Wl2, Wr2):
    def body(aggp_ref, cntp_ref, x_ref, wl1_ref, bl1_ref, wr1_ref,
             wl2_ref, wr2_ref, p1a_ref, p1b_ref, q_ref, cnti_ref):
        agg = aggp_ref[0] + aggp_ref[1]
        cnt = cntp_ref[0] + cntp_ref[1]
        cnti = 1.0 / jnp.maximum(cnt, 1.0)
        mean = agg * cnti
        dn = (((1,), (1,)), ((), ()))
        out1 = (lax.dot_general(mean, wl1_ref[...], dn,
                                preferred_element_type=jnp.float32)
                + bl1_ref[...]
                + lax.dot_general(x_ref[...], wr1_ref[...], dn,
                                  preferred_element_type=jnp.float32))
        rn = jnp.sqrt(jnp.sum(out1 * out1, axis=1, keepdims=True))
        out1 = out1 / jnp.maximum(rn, 1e-12)
        h1 = _elu(out1)
        p1 = lax.dot_general(h1, wl2_ref[...], dn,
                             preferred_element_type=jnp.float32)
        q = lax.dot_general(h1, wr2_ref[...], dn,
                            preferred_element_type=jnp.float32)
        p1a_ref[...] = p1[:, :F]
        p1b_ref[...] = p1[:, F:]
        q_ref[...] = q
        cnti_ref[...] = cnti

    full = lambda shape: pl.BlockSpec(shape, lambda i: tuple(0 for _ in shape))
    return pl.pallas_call(
        body,
        grid=(GRID_N,),
        in_specs=[
            pl.BlockSpec((NC, BN, F), lambda i: (0, i, 0)),
            pl.BlockSpec((NC, BN, 1), lambda i: (0, i, 0)),
            pl.BlockSpec((BN, F), lambda i: (i, 0)),
            full((4 * F, F)),
            full((1, 4 * F)),
            full((4 * F, F)),
            full((2 * F, 4 * F)),
            full((2 * F, 4 * F)),
        ],
        out_specs=[
            pl.BlockSpec((BN, F), lambda i: (i, 0)),
            pl.BlockSpec((BN, F), lambda i: (i, 0)),
            pl.BlockSpec((BN, 2 * F), lambda i: (i, 0)),
            pl.BlockSpec((BN, 1), lambda i: (i, 0)),
        ],
        out_shape=[
            jax.ShapeDtypeStruct((N, F), jnp.float32),
            jax.ShapeDtypeStruct((N, F), jnp.float32),
            jax.ShapeDtypeStruct((N, 2 * F), jnp.float32),
            jax.ShapeDtypeStruct((N, 1), jnp.float32),
        ],
    )(aggp, cntp3, x, Wl1, bl1.reshape(1, 4 * F), Wr1, Wl2, Wr2)


def _dense2(agg2p, cnti, q, bl2):
    def body(a2_ref, cnti_ref, q_ref, bl2_ref, gm_ref):
        agg2 = jnp.concatenate([a2_ref[0], a2_ref[1]], axis=1)
        out2 = agg2 * cnti_ref[...] + bl2_ref[...] + q_ref[...]
        rn = jnp.sqrt(jnp.sum(out2 * out2, axis=1, keepdims=True))
        out2 = out2 / jnp.maximum(rn, 1e-12)
        h2 = _elu(out2)
        m = jnp.max(h2, axis=0, keepdims=True)

        @pl.when(pl.program_id(0) == 0)
        def _():
            gm_ref[...] = jnp.full((1, 2 * F), -jnp.inf, jnp.float32)

        gm_ref[...] = jnp.maximum(gm_ref[...], m)

    return pl.pallas_call(
        body,
        grid=(GRID_N,),
        in_specs=[
            pl.BlockSpec((NC, BN, F), lambda i: (0, i, 0)),
            pl.BlockSpec((BN, 1), lambda i: (i, 0)),
            pl.BlockSpec((BN, 2 * F), lambda i: (i, 0)),
            pl.BlockSpec((1, 2 * F), lambda i: (0, 0)),
        ],
        out_specs=pl.BlockSpec((1, 2 * F), lambda i: (0, 0)),
        out_shape=jax.ShapeDtypeStruct((1, 2 * F), jnp.float32),
    )(agg2p, cnti, q, bl2.reshape(1, 2 * F))


def _head(gmT, thetaT, Wfc, bfc, Wn1, bn1, Wn2, bn2, Wn3, bn3, Wn4, bn4,
          Wno, bno):
    def body(gm_ref, th_ref, wfc_ref, bfc_ref, w1_ref, b1_ref, w2_ref, b2_ref,
             w3_ref, b3_ref, w4_ref, b4_ref, wo_ref, bo_ref, out_ref):
        def mv(w_ref, v, b_ref):
            return jnp.dot(w_ref[...], v,
                           preferred_element_type=jnp.float32) + b_ref[...]

        g = _elu(gm_ref[...])
        g2 = mv(wfc_ref, g, bfc_ref)
        c = _elu(mv(w1_ref, g2, b1_ref))
        c = _elu(mv(w2_ref, c, b2_ref))
        c = _elu(mv(w3_ref, c, b3_ref))
        c = _elu(mv(w4_ref, c, b4_ref))
        st = mv(wo_ref, c, bo_ref)
        mu = st[:TD, :]
        ls = st[TD:, :]
        z = (th_ref[...] - mu) * jnp.exp(-ls)
        const = (jnp.sum(ls) + 0.5 * TD * jnp.log(2.0 * jnp.pi))
        out_ref[...] = -0.5 * jnp.sum(z * z, axis=0, keepdims=True) - const

    args = [gmT, thetaT, Wfc, bfc.reshape(F, 1), Wn1, bn1.reshape(H, 1),
            Wn2, bn2.reshape(H, 1), Wn3, bn3.reshape(H, 1),
            Wn4, bn4.reshape(H, 1), Wno, bno.reshape(2 * TD, 1)]
    return pl.pallas_call(
        body,
        out_shape=jax.ShapeDtypeStruct((1, B), jnp.float32),
    )(*args)


def kernel(theta, x, edge_index, Wl1, bl1, Wr1, Wl2, bl2, Wr2, Wfc, bfc,
           Wn1, bn1, Wn2, bn2, Wn3, bn3, Wn4, bn4, Wno, bno):
    src = edge_index[0].astype(jnp.int32)
    dst = edge_index[1].astype(jnp.int32)
    pad = EP - E
    srcp = jnp.concatenate([src, jnp.zeros((pad,), jnp.int32)])
    dstp = jnp.concatenate([dst, jnp.full((pad,), N, jnp.int32)])
    src2 = srcp
    dst2 = dstp
    zagg = jnp.zeros((NSLICE, F), jnp.float32)
    zcnt = jnp.zeros((NSLICE,), jnp.float32)

    aggp, cntp = _sc_pass1(x, src2, dst2, zagg, zcnt)
    p1a, p1b, q, cnti = _dense1(aggp, cntp.reshape(NC, NP, 1), x,
                                Wl1, bl1, Wr1, Wl2, Wr2)
    agg2p = _sc_pass2(p1a, p1b, src2, dst2, zagg)
    gm = _dense2(agg2p, cnti, q, bl2)
    logp = _head(gm.reshape(2 * F, 1), theta.T, Wfc, bfc, Wn1, bn1,
                 Wn2, bn2, Wn3, bn3, Wn4, bn4, Wno, bno)
    return logp[0]


# speedup vs baseline: 28.6439x; 1.2188x over previous
import functools

import jax
import jax.numpy as jnp
from jax import lax
from jax.experimental import pallas as pl
from jax.experimental.pallas import tpu as pltpu
from jax.experimental.pallas import tpu_sc as plsc

N = 100000
E = 3200000
F = 16
B = 1024
TD = 8
H = 128

NC = 2
NS = 16
NW = NC * NS
NSLICE = 6272
NP = NS * NSLICE
EP_ROWS = 25088
EP = EP_ROWS * 128
CE1 = 512
CE2 = 784
ET1 = EP // NW
ET2 = EP // NS

BN = 2000
GRID_N = N // BN


def _elu(v):
    return jnp.where(v > 0, v, jnp.exp(v) - 1.0)


def _sc_mesh():
    return plsc.VectorSubcoreMesh(core_axis_name="c", subcore_axis_name="s",
                                  num_cores=NC, num_subcores=NS)


_SC_PARAMS = pltpu.CompilerParams(use_tc_tiling_on_sc=False)


def _sc_pass1(xt, srcp, dstp, zagg, zcnt):
    mesh = _sc_mesh()

    @functools.partial(
        pl.kernel,
        out_type=[
            jax.ShapeDtypeStruct((NC, NP, F), jnp.float32),
            jax.ShapeDtypeStruct((NC, NP), jnp.float32),
        ],
        mesh=mesh,
        compiler_params=_SC_PARAMS,
        scratch_types=[
            pltpu.VMEM((CE1,), jnp.int32),
            pltpu.VMEM((CE1,), jnp.int32),
            pltpu.VMEM((CE1,), jnp.int32),
            pltpu.VMEM((CE1,), jnp.int32),
            pltpu.VMEM((CE1, F), jnp.float32),
            pltpu.VMEM((CE1, F), jnp.float32),
            pltpu.VMEM((CE1,), jnp.float32),
        ] + [pltpu.SemaphoreType.DMA] * 8 + [
            pltpu.VMEM_SHARED((NP, F), jnp.float32),
            pltpu.VMEM_SHARED((NP,), jnp.float32),
        ],
    )
    def k(xt_hbm, src_hbm, dst_hbm, zagg_hbm, zcnt_hbm, aggp_hbm, cntp_hbm,
          sx0, sx1, dx0, dx1, rows0, rows1, ones,
          si0, si1, sg0, sg1, ss0, ss1, sc0, sc1, agg_sh, cnt_sh):
        cid = lax.axis_index("c")
        sid = lax.axis_index("s")
        wid = cid * NS + sid
        for t in range(CE1 // 16):
            ones[pl.ds(t * 16, 16)] = jnp.ones((16,), jnp.float32)
        sl = pl.ds(sid * NSLICE, NSLICE)
        pltpu.sync_copy(zagg_hbm, agg_sh.at[sl])
        pltpu.sync_copy(zcnt_hbm, cnt_sh.at[sl])
        plsc.subcore_barrier()

        nch = (EP // CE1) // NW
        e0b = wid * nch * CE1

        def load_idx(c, sx, dx, sem):
            pltpu.async_copy(src_hbm.at[pl.ds(e0b + c * CE1, CE1)], sx, sem)
            pltpu.async_copy(dst_hbm.at[pl.ds(e0b + c * CE1, CE1)], dx, sem)

        def wait_idx(c, sx, dx, sem):
            pltpu.make_async_copy(src_hbm.at[pl.ds(e0b + c * CE1, CE1)], sx, sem).wait()
            pltpu.make_async_copy(dst_hbm.at[pl.ds(e0b + c * CE1, CE1)], dx, sem).wait()

        load_idx(0, sx0, dx0, si0)

        @pl.loop(0, nch, step=2)
        def _(i):
            wait_idx(i, sx0, dx0, si0)
            pltpu.async_copy(xt_hbm.at[sx0], rows0, sg0)

            @pl.when(i > 0)
            def _():
                pltpu.make_async_copy(rows1, agg_sh.at[dx1], ss1).wait()
                pltpu.make_async_copy(ones, cnt_sh.at[dx1], sc1).wait()

            load_idx(i + 1, sx1, dx1, si1)
            pltpu.make_async_copy(xt_hbm.at[sx0], rows0, sg0).wait()
            pltpu.async_copy(rows0, agg_sh.at[dx0], ss0, add=True)
            pltpu.async_copy(ones, cnt_sh.at[dx0], sc0, add=True)

            wait_idx(i + 1, sx1, dx1, si1)
            pltpu.async_copy(xt_hbm.at[sx1], rows1, sg1)
            pltpu.make_async_copy(rows0, agg_sh.at[dx0], ss0).wait()
            pltpu.make_async_copy(ones, cnt_sh.at[dx0], sc0).wait()

            @pl.when(i + 2 < nch)
            def _():
                load_idx(i + 2, sx0, dx0, si0)

            pltpu.make_async_copy(xt_hbm.at[sx1], rows1, sg1).wait()
            pltpu.async_copy(rows1, agg_sh.at[dx1], ss1, add=True)
            pltpu.async_copy(ones, cnt_sh.at[dx1], sc1, add=True)

        pltpu.make_async_copy(rows1, agg_sh.at[dx1], ss1).wait()
        pltpu.make_async_copy(ones, cnt_sh.at[dx1], sc1).wait()
        plsc.subcore_barrier()
        pltpu.sync_copy(agg_sh.at[sl], aggp_hbm.at[cid].at[sl])
        pltpu.sync_copy(cnt_sh.at[sl], cntp_hbm.at[cid].at[sl])

    return k(xt, srcp, dstp, zagg, zcnt)


def _sc_pass2(ta, tb, srcp, dstp, zagg):
    mesh = _sc_mesh()

    @functools.partial(
        pl.kernel,
        out_type=jax.ShapeDtypeStruct((NC, NP, F), jnp.float32),
        mesh=mesh,
        compiler_params=_SC_PARAMS,
        scratch_types=[
            pltpu.VMEM((CE2,), jnp.int32),
            pltpu.VMEM((CE2,), jnp.int32),
            pltpu.VMEM((CE2,), jnp.int32),
            pltpu.VMEM((CE2,), jnp.int32),
            pltpu.VMEM((CE2, F), jnp.float32),
            pltpu.VMEM((CE2, F), jnp.float32),
        ] + [pltpu.SemaphoreType.DMA] * 6 + [
            pltpu.VMEM_SHARED((NP, F), jnp.float32),
        ],
    )
    def k(ta_hbm, tb_hbm, src_hbm, dst_hbm, zagg_hbm, aggp_hbm,
          sx0, sx1, dx0, dx1, rows0, rows1, si0, si1, sg0, sg1, ss0, ss1, agg_sh):
        cid = lax.axis_index("c")
        sid = lax.axis_index("s")
        sl = pl.ds(sid * NSLICE, NSLICE)
        pltpu.sync_copy(zagg_hbm, agg_sh.at[sl])
        plsc.subcore_barrier()

        def edge_loop(tab_hbm):
            nch = (EP // CE2) // NS
            e0b = sid * nch * CE2

            def load_idx(c, sx, dx, sem):
                pltpu.async_copy(src_hbm.at[pl.ds(e0b + c * CE2, CE2)], sx, sem)
                pltpu.async_copy(dst_hbm.at[pl.ds(e0b + c * CE2, CE2)], dx, sem)

            def wait_idx(c, sx, dx, sem):
                pltpu.make_async_copy(src_hbm.at[pl.ds(e0b + c * CE2, CE2)], sx, sem).wait()
                pltpu.make_async_copy(dst_hbm.at[pl.ds(e0b + c * CE2, CE2)], dx, sem).wait()

            load_idx(0, sx0, dx0, si0)

            @pl.loop(0, nch, step=2)
            def _(i):
                wait_idx(i, sx0, dx0, si0)
                pltpu.async_copy(tab_hbm.at[sx0], rows0, sg0)

                @pl.when(i > 0)
                def _():
                    pltpu.make_async_copy(rows1, agg_sh.at[dx1], ss1).wait()

                load_idx(i + 1, sx1, dx1, si1)
                pltpu.make_async_copy(tab_hbm.at[sx0], rows0, sg0).wait()
                pltpu.async_copy(rows0, agg_sh.at[dx0], ss0, add=True)

                wait_idx(i + 1, sx1, dx1, si1)
                pltpu.async_copy(tab_hbm.at[sx1], rows1, sg1)
                pltpu.make_async_copy(rows0, agg_sh.at[dx0], ss0).wait()

                @pl.when(i + 2 < nch)
                def _():
                    load_idx(i + 2, sx0, dx0, si0)

                pltpu.make_async_copy(tab_hbm.at[sx1], rows1, sg1).wait()
                pltpu.async_copy(rows1, agg_sh.at[dx1], ss1, add=True)

            pltpu.make_async_copy(rows1, agg_sh.at[dx1], ss1).wait()

        @pl.when(cid == 0)
        def _():
            edge_loop(ta_hbm)

        @pl.when(cid == 1)
        def _():
            edge_loop(tb_hbm)

        plsc.subcore_barrier()
        pltpu.sync_copy(agg_sh.at[sl], aggp_hbm.at[cid].at[sl])

    return k(ta, tb, srcp, dstp, zagg)


def _dense1(aggp, cntp3, x, Wl1, bl1, Wr1, Wl2, Wr2):
    def body(aggp_ref, cntp_ref, x_ref, wl1_ref, bl1_ref, wr1_ref,
             wl2_ref, wr2_ref, p1a_ref, p1b_ref, q_ref, cnti_ref):
        agg = aggp_ref[0] + aggp_ref[1]
        cnt = cntp_ref[0] + cntp_ref[1]
        cnti = 1.0 / jnp.maximum(cnt, 1.0)
        mean = agg * cnti
        dn = (((1,), (1,)), ((), ()))
        out1 = (lax.dot_general(mean, wl1_ref[...], dn,
                                preferred_element_type=jnp.float32)
                + bl1_ref[...]
                + lax.dot_general(x_ref[...], wr1_ref[...], dn,
                                  preferred_element_type=jnp.float32))
        rn = jnp.sqrt(jnp.sum(out1 * out1, axis=1, keepdims=True))
        out1 = out1 / jnp.maximum(rn, 1e-12)
        h1 = _elu(out1)
        p1 = lax.dot_general(h1, wl2_ref[...], dn,
                             preferred_element_type=jnp.float32)
        q = lax.dot_general(h1, wr2_ref[...], dn,
                            preferred_element_type=jnp.float32)
        p1a_ref[...] = p1[:, :F]
        p1b_ref[...] = p1[:, F:]
        q_ref[...] = q
        cnti_ref[...] = cnti

    full = lambda shape: pl.BlockSpec(shape, lambda i: tuple(0 for _ in shape))
    return pl.pallas_call(
        body,
        grid=(GRID_N,),
        in_specs=[
            pl.BlockSpec((NC, BN, F), lambda i: (0, i, 0)),
            pl.BlockSpec((NC, BN, 1), lambda i: (0, i, 0)),
            pl.BlockSpec((BN, F), lambda i: (i, 0)),
            full((4 * F, F)),
            full((1, 4 * F)),
            full((4 * F, F)),
            full((2 * F, 4 * F)),
            full((2 * F, 4 * F)),
        ],
        out_specs=[
            pl.BlockSpec((BN, F), lambda i: (i, 0)),
            pl.BlockSpec((BN, F), lambda i: (i, 0)),
            pl.BlockSpec((BN, 2 * F), lambda i: (i, 0)),
            pl.BlockSpec((BN, 1), lambda i: (i, 0)),
        ],
        out_shape=[
            jax.ShapeDtypeStruct((N, F), jnp.float32),
            jax.ShapeDtypeStruct((N, F), jnp.float32),
            jax.ShapeDtypeStruct((N, 2 * F), jnp.float32),
            jax.ShapeDtypeStruct((N, 1), jnp.float32),
        ],
    )(aggp, cntp3, x, Wl1, bl1.reshape(1, 4 * F), Wr1, Wl2, Wr2)


def _dense2(agg2p, cnti, q, bl2):
    def body(a2_ref, cnti_ref, q_ref, bl2_ref, gm_ref):
        agg2 = jnp.concatenate([a2_ref[0], a2_ref[1]], axis=1)
        out2 = agg2 * cnti_ref[...] + bl2_ref[...] + q_ref[...]
        rn = jnp.sqrt(jnp.sum(out2 * out2, axis=1, keepdims=True))
        out2 = out2 / jnp.maximum(rn, 1e-12)
        h2 = _elu(out2)
        m = jnp.max(h2, axis=0, keepdims=True)

        @pl.when(pl.program_id(0) == 0)
        def _():
            gm_ref[...] = jnp.full((1, 2 * F), -jnp.inf, jnp.float32)

        gm_ref[...] = jnp.maximum(gm_ref[...], m)

    return pl.pallas_call(
        body,
        grid=(GRID_N,),
        in_specs=[
            pl.BlockSpec((NC, BN, F), lambda i: (0, i, 0)),
            pl.BlockSpec((BN, 1), lambda i: (i, 0)),
            pl.BlockSpec((BN, 2 * F), lambda i: (i, 0)),
            pl.BlockSpec((1, 2 * F), lambda i: (0, 0)),
        ],
        out_specs=pl.BlockSpec((1, 2 * F), lambda i: (0, 0)),
        out_shape=jax.ShapeDtypeStruct((1, 2 * F), jnp.float32),
    )(agg2p, cnti, q, bl2.reshape(1, 2 * F))


def _head(gmT, thetaT, Wfc, bfc, Wn1, bn1, Wn2, bn2, Wn3, bn3, Wn4, bn4,
          Wno, bno):
    def body(gm_ref, th_ref, wfc_ref, bfc_ref, w1_ref, b1_ref, w2_ref, b2_ref,
             w3_ref, b3_ref, w4_ref, b4_ref, wo_ref, bo_ref, out_ref):
        def mv(w_ref, v, b_ref):
            return jnp.dot(w_ref[...], v,
                           preferred_element_type=jnp.float32) + b_ref[...]

        g = _elu(gm_ref[...])
        g2 = mv(wfc_ref, g, bfc_ref)
        c = _elu(mv(w1_ref, g2, b1_ref))
        c = _elu(mv(w2_ref, c, b2_ref))
        c = _elu(mv(w3_ref, c, b3_ref))
        c = _elu(mv(w4_ref, c, b4_ref))
        st = mv(wo_ref, c, bo_ref)
        mu = st[:TD, :]
        ls = st[TD:, :]
        z = (th_ref[...] - mu) * jnp.exp(-ls)
        const = (jnp.sum(ls) + 0.5 * TD * jnp.log(2.0 * jnp.pi))
        out_ref[...] = -0.5 * jnp.sum(z * z, axis=0, keepdims=True) - const

    args = [gmT, thetaT, Wfc, bfc.reshape(F, 1), Wn1, bn1.reshape(H, 1),
            Wn2, bn2.reshape(H, 1), Wn3, bn3.reshape(H, 1),
            Wn4, bn4.reshape(H, 1), Wno, bno.reshape(2 * TD, 1)]
    return pl.pallas_call(
        body,
        out_shape=jax.ShapeDtypeStruct((1, B), jnp.float32),
    )(*args)


def kernel(theta, x, edge_index, Wl1, bl1, Wr1, Wl2, bl2, Wr2, Wfc, bfc,
           Wn1, bn1, Wn2, bn2, Wn3, bn3, Wn4, bn4, Wno, bno):
    src = edge_index[0].astype(jnp.int32)
    dst = edge_index[1].astype(jnp.int32)
    pad = EP - E
    srcp = jnp.concatenate([src, jnp.zeros((pad,), jnp.int32)])
    dstp = jnp.concatenate([dst, jnp.full((pad,), N, jnp.int32)])
    zagg = jnp.zeros((NSLICE, F), jnp.float32)
    zcnt = jnp.zeros((NSLICE,), jnp.float32)

    aggp, cntp = _sc_pass1(x, srcp, dstp, zagg, zcnt)
    p1a, p1b, q, cnti = _dense1(aggp, cntp.reshape(NC, NP, 1), x,
                                Wl1, bl1, Wr1, Wl2, Wr2)
    agg2p = _sc_pass2(p1a, p1b, srcp, dstp, zagg)
    gm = _dense2(agg2p, cnti, q, bl2)
    logp = _head(gm.reshape(2 * F, 1), theta.T, Wfc, bfc, Wn1, bn1,
                 Wn2, bn2, Wn3, bn3, Wn4, bn4, Wno, bno)
    return logp[0]


# speedup vs baseline: 37.2298x; 1.2997x over previous
import functools

import jax
import jax.numpy as jnp
from jax import lax
from jax.experimental import pallas as pl
from jax.experimental.pallas import tpu as pltpu
from jax.experimental.pallas import tpu_sc as plsc

N = 100000
E = 3200000
F = 16
B = 1024
TD = 8
H = 128

NC = 2
NS = 16
NW = NC * NS
NSLICE = 6272
NP = NS * NSLICE
EP_ROWS = 25088
EP = EP_ROWS * 128
CE1 = 512
CE2 = 784
ET1 = EP // NW
ET2 = EP // NS

BN = 2000
GRID_N = N // BN


def _elu(v):
    return jnp.where(v > 0, v, jnp.exp(v) - 1.0)


def _sc_mesh():
    return plsc.VectorSubcoreMesh(core_axis_name="c", subcore_axis_name="s",
                                  num_cores=NC, num_subcores=NS)


_SC_PARAMS = pltpu.CompilerParams(use_tc_tiling_on_sc=False)


def _sc_pass1(xt, srcp, dstp, zagg, zcnt):
    mesh = _sc_mesh()

    @functools.partial(
        pl.kernel,
        out_type=[
            jax.ShapeDtypeStruct((NC, NP, F), jnp.float32),
            jax.ShapeDtypeStruct((NC, NP), jnp.float32),
        ],
        mesh=mesh,
        compiler_params=_SC_PARAMS,
        scratch_types=[
            pltpu.VMEM((CE1,), jnp.int32),
            pltpu.VMEM((CE1,), jnp.int32),
            pltpu.VMEM((CE1,), jnp.int32),
            pltpu.VMEM((CE1,), jnp.int32),
            pltpu.VMEM((CE1, F), jnp.float32),
            pltpu.VMEM((CE1, F), jnp.float32),
            pltpu.VMEM((CE1,), jnp.float32),
        ] + [pltpu.SemaphoreType.DMA] * 8 + [
            pltpu.VMEM_SHARED((NP, F), jnp.float32),
            pltpu.VMEM_SHARED((NP,), jnp.float32),
        ],
    )
    def k(xt_hbm, src_hbm, dst_hbm, zagg_hbm, zcnt_hbm, aggp_hbm, cntp_hbm,
          sx0, sx1, dx0, dx1, rows0, rows1, ones,
          si0, si1, sg0, sg1, ss0, ss1, sc0, sc1, agg_sh, cnt_sh):
        cid = lax.axis_index("c")
        sid = lax.axis_index("s")
        wid = cid * NS + sid
        for t in range(CE1 // 16):
            ones[pl.ds(t * 16, 16)] = jnp.ones((16,), jnp.float32)
        sl = pl.ds(sid * NSLICE, NSLICE)
        pltpu.sync_copy(zagg_hbm, agg_sh.at[sl])
        pltpu.sync_copy(zcnt_hbm, cnt_sh.at[sl])
        plsc.subcore_barrier()

        nch = (EP // CE1) // NW
        e0b = wid * nch * CE1

        def load_idx(c, sx, dx, sem):
            pltpu.async_copy(src_hbm.at[pl.ds(e0b + c * CE1, CE1)], sx, sem)
            pltpu.async_copy(dst_hbm.at[pl.ds(e0b + c * CE1, CE1)], dx, sem)

        def wait_idx(c, sx, dx, sem):
            pltpu.make_async_copy(src_hbm.at[pl.ds(e0b + c * CE1, CE1)], sx, sem).wait()
            pltpu.make_async_copy(dst_hbm.at[pl.ds(e0b + c * CE1, CE1)], dx, sem).wait()

        load_idx(0, sx0, dx0, si0)

        @pl.loop(0, nch, step=2)
        def _(i):
            wait_idx(i, sx0, dx0, si0)
            pltpu.async_copy(xt_hbm.at[sx0], rows0, sg0)

            @pl.when(i > 0)
            def _():
                pltpu.make_async_copy(rows1, agg_sh.at[dx1], ss1).wait()
                pltpu.make_async_copy(ones, cnt_sh.at[dx1], sc1).wait()

            load_idx(i + 1, sx1, dx1, si1)
            pltpu.make_async_copy(xt_hbm.at[sx0], rows0, sg0).wait()
            pltpu.async_copy(rows0, agg_sh.at[dx0], ss0, add=True)
            pltpu.async_copy(ones, cnt_sh.at[dx0], sc0, add=True)

            wait_idx(i + 1, sx1, dx1, si1)
            pltpu.async_copy(xt_hbm.at[sx1], rows1, sg1)
            pltpu.make_async_copy(rows0, agg_sh.at[dx0], ss0).wait()
            pltpu.make_async_copy(ones, cnt_sh.at[dx0], sc0).wait()

            @pl.when(i + 2 < nch)
            def _():
                load_idx(i + 2, sx0, dx0, si0)

            pltpu.make_async_copy(xt_hbm.at[sx1], rows1, sg1).wait()
            pltpu.async_copy(rows1, agg_sh.at[dx1], ss1, add=True)
            pltpu.async_copy(ones, cnt_sh.at[dx1], sc1, add=True)

        pltpu.make_async_copy(rows1, agg_sh.at[dx1], ss1).wait()
        pltpu.make_async_copy(ones, cnt_sh.at[dx1], sc1).wait()
        plsc.subcore_barrier()
        pltpu.sync_copy(agg_sh.at[sl], aggp_hbm.at[cid].at[sl])
        pltpu.sync_copy(cnt_sh.at[sl], cntp_hbm.at[cid].at[sl])

    return k(xt, srcp, dstp, zagg, zcnt)


def _sc_pass2(ta, tb, srcp, dstp, zagg):
    mesh = _sc_mesh()

    @functools.partial(
        pl.kernel,
        out_type=jax.ShapeDtypeStruct((NC, NP, F), jnp.float32),
        mesh=mesh,
        compiler_params=_SC_PARAMS,
        scratch_types=[
            pltpu.VMEM((CE2,), jnp.int32),
            pltpu.VMEM((CE2,), jnp.int32),
            pltpu.VMEM((CE2,), jnp.int32),
            pltpu.VMEM((CE2,), jnp.int32),
            pltpu.VMEM((CE2, F), jnp.float32),
            pltpu.VMEM((CE2, F), jnp.float32),
        ] + [pltpu.SemaphoreType.DMA] * 6 + [
            pltpu.VMEM_SHARED((NP, F), jnp.float32),
        ],
    )
    def k(ta_hbm, tb_hbm, src_hbm, dst_hbm, zagg_hbm, aggp_hbm,
          sx0, sx1, dx0, dx1, rows0, rows1, si0, si1, sg0, sg1, ss0, ss1, agg_sh):
        cid = lax.axis_index("c")
        sid = lax.axis_index("s")
        sl = pl.ds(sid * NSLICE, NSLICE)
        pltpu.sync_copy(zagg_hbm, agg_sh.at[sl])
        plsc.subcore_barrier()

        def edge_loop(tab_hbm):
            nch = (EP // CE2) // NS
            e0b = sid * nch * CE2

            def load_idx(c, sx, dx, sem):
                pltpu.async_copy(src_hbm.at[pl.ds(e0b + c * CE2, CE2)], sx, sem)
                pltpu.async_copy(dst_hbm.at[pl.ds(e0b + c * CE2, CE2)], dx, sem)

            def wait_idx(c, sx, dx, sem):
                pltpu.make_async_copy(src_hbm.at[pl.ds(e0b + c * CE2, CE2)], sx, sem).wait()
                pltpu.make_async_copy(dst_hbm.at[pl.ds(e0b + c * CE2, CE2)], dx, sem).wait()

            load_idx(0, sx0, dx0, si0)

            @pl.loop(0, nch, step=2)
            def _(i):
                wait_idx(i, sx0, dx0, si0)
                pltpu.async_copy(tab_hbm.at[sx0], rows0, sg0)

                @pl.when(i > 0)
                def _():
                    pltpu.make_async_copy(rows1, agg_sh.at[dx1], ss1).wait()

                load_idx(i + 1, sx1, dx1, si1)
                pltpu.make_async_copy(tab_hbm.at[sx0], rows0, sg0).wait()
                pltpu.async_copy(rows0, agg_sh.at[dx0], ss0, add=True)

                wait_idx(i + 1, sx1, dx1, si1)
                pltpu.async_copy(tab_hbm.at[sx1], rows1, sg1)
                pltpu.make_async_copy(rows0, agg_sh.at[dx0], ss0).wait()

                @pl.when(i + 2 < nch)
                def _():
                    load_idx(i + 2, sx0, dx0, si0)

                pltpu.make_async_copy(tab_hbm.at[sx1], rows1, sg1).wait()
                pltpu.async_copy(rows1, agg_sh.at[dx1], ss1, add=True)

            pltpu.make_async_copy(rows1, agg_sh.at[dx1], ss1).wait()

        @pl.when(cid == 0)
        def _():
            edge_loop(ta_hbm)

        @pl.when(cid == 1)
        def _():
            edge_loop(tb_hbm)

        plsc.subcore_barrier()
        pltpu.sync_copy(agg_sh.at[sl], aggp_hbm.at[cid].at[sl])

    return k(ta, tb, srcp, dstp, zagg)


NP8 = NP // 8
BNP = 256
NPK_LAST = N // 8 - (NP8 // BNP - 1) * BNP


def _dense1(aggp8, cnt8, xp, KWl1, bl1t, KWr1, KW2a, KW2b, KWra, KWrb, G512):
    def body(ap_ref, cp_ref, xp_ref, kwl1_ref, bl1_ref, kwr1_ref,
             kw2a_ref, kw2b_ref, kwra_ref, kwrb_ref, g512_ref,
             p1a_ref, p1b_ref, qa_ref, qb_ref, cs_ref):
        agg = ap_ref[0] + ap_ref[1]
        cs = 1.0 / jnp.maximum(cp_ref[0] + cp_ref[1], 1.0)
        mean = agg * cs
        o = (jnp.dot(mean, kwl1_ref[...], preferred_element_type=jnp.float32)
             + bl1_ref[...]
             + jnp.dot(xp_ref[...], kwr1_ref[...],
                       preferred_element_type=jnp.float32))
        ss = jnp.dot(o * o, g512_ref[...], preferred_element_type=jnp.float32)
        o = o / jnp.maximum(jnp.sqrt(ss), 1e-12)
        h = _elu(o)
        p1a_ref[...] = jnp.dot(h, kw2a_ref[...],
                               preferred_element_type=jnp.float32)
        p1b_ref[...] = jnp.dot(h, kw2b_ref[...],
                               preferred_element_type=jnp.float32)
        qa_ref[...] = jnp.dot(h, kwra_ref[...],
                              preferred_element_type=jnp.float32)
        qb_ref[...] = jnp.dot(h, kwrb_ref[...],
                              preferred_element_type=jnp.float32)
        cs_ref[...] = cs

    full = lambda shape: pl.BlockSpec(shape, lambda i: tuple(0 for _ in shape))
    bs = pl.BlockSpec((BNP, 128), lambda i: (i, 0))
    return pl.pallas_call(
        body,
        grid=(NP8 // BNP,),
        in_specs=[
            pl.BlockSpec((NC, BNP, 128), lambda i: (0, i, 0)),
            pl.BlockSpec((NC, BNP, 128), lambda i: (0, i, 0)),
            bs,
            full((128, 512)), full((1, 512)), full((128, 512)),
            full((512, 128)), full((512, 128)),
            full((512, 128)), full((512, 128)),
            full((512, 512)),
        ],
        out_specs=[bs, bs, bs, bs, bs],
        out_shape=[jax.ShapeDtypeStruct((NP8, 128), jnp.float32)] * 5,
    )(aggp8, cnt8, xp, KWl1, bl1t, KWr1, KW2a, KW2b, KWra, KWrb, G512)


def _dense2(a2p8, cs8, qa8, qb8, bl2at, bl2bt, G128, theta,
            Wfc, bfc, Wn1, bn1, Wn2, bn2, Wn3, bn3, Wn4, bn4, Wno, bno):
    grid_n = NP8 // BNP
    dnT = (((1,), (1,)), ((), ()))

    def body(a2_ref, cs_ref, qa_ref, qb_ref, b2a_ref, b2b_ref, g128_ref,
             th_ref, wfc_ref, bfc_ref, w1_ref, b1_ref, w2_ref, b2_ref,
             w3_ref, b3_ref, w4_ref, b4_ref, wo_ref, bo_ref,
             gma_ref, gmb_ref, lp_ref):
        cs = cs_ref[...]
        oa = a2_ref[0] * cs + b2a_ref[...] + qa_ref[...]
        ob = a2_ref[1] * cs + b2b_ref[...] + qb_ref[...]
        ss = (jnp.dot(oa * oa, g128_ref[...],
                      preferred_element_type=jnp.float32)
              + jnp.dot(ob * ob, g128_ref[...],
                        preferred_element_type=jnp.float32))
        inv = 1.0 / jnp.maximum(jnp.sqrt(ss), 1e-12)
        ha = _elu(oa * inv)
        hb = _elu(ob * inv)
        rowid = lax.broadcasted_iota(jnp.int32, (BNP, 1), 0)
        lim = jnp.where(pl.program_id(0) == grid_n - 1, NPK_LAST, BNP)
        ninf = jnp.float32(-jnp.inf)
        ma = jnp.max(jnp.where(rowid < lim, ha, ninf), axis=0, keepdims=True)
        mb = jnp.max(jnp.where(rowid < lim, hb, ninf), axis=0, keepdims=True)

        @pl.when(pl.program_id(0) == 0)
        def _():
            gma_ref[...] = jnp.full((1, 128), -jnp.inf, jnp.float32)
            gmb_ref[...] = jnp.full((1, 128), -jnp.inf, jnp.float32)

        gma_ref[...] = jnp.maximum(gma_ref[...], ma)
        gmb_ref[...] = jnp.maximum(gmb_ref[...], mb)

        @pl.when(pl.program_id(0) == grid_n - 1)
        def _():
            va = gma_ref[...]
            vb = gmb_ref[...]
            ga = va[:, 0:F]
            gb = vb[:, 0:F]
            for r in range(1, 8):
                ga = jnp.maximum(ga, va[:, r * F:(r + 1) * F])
                gb = jnp.maximum(gb, vb[:, r * F:(r + 1) * F])
            g = _elu(jnp.concatenate([ga, gb], axis=1))
            gf = lax.dot_general(g, wfc_ref[...], dnT,
                                 preferred_element_type=jnp.float32) + bfc_ref[...]
            c = gf
            for w_ref, b_ref in ((w1_ref, b1_ref), (w2_ref, b2_ref),
                                 (w3_ref, b3_ref), (w4_ref, b4_ref)):
                c = _elu(lax.dot_general(c, w_ref[...], dnT,
                                         preferred_element_type=jnp.float32)
                         + b_ref[...])
            st = lax.dot_general(c, wo_ref[...], dnT,
                                 preferred_element_type=jnp.float32) + bo_ref[...]
            mu = st[:, :TD]
            ls = st[:, TD:]
            z = (th_ref[...] - mu) * jnp.exp(-ls)
            const = jnp.sum(ls) + 0.5 * TD * jnp.log(2.0 * jnp.pi)
            lp_ref[...] = (-0.5 * jnp.sum(z * z, axis=1, keepdims=True)
                           - const)

    full = lambda shape: pl.BlockSpec(shape, lambda i: tuple(0 for _ in shape))
    bs = pl.BlockSpec((BNP, 128), lambda i: (i, 0))
    _, _, lp = pl.pallas_call(
        body,
        grid=(grid_n,),
        in_specs=[
            pl.BlockSpec((NC, BNP, 128), lambda i: (0, i, 0)),
            bs, bs, bs,
            full((1, 128)), full((1, 128)), full((128, 128)),
            full((B, TD)),
            full((F, 2 * F)), full((1, F)),
            full((H, F)), full((1, H)),
            full((H, H)), full((1, H)),
            full((H, H)), full((1, H)),
            full((H, H)), full((1, H)),
            full((2 * TD, H)), full((1, 2 * TD)),
        ],
        out_specs=[
            pl.BlockSpec((1, 128), lambda i: (0, 0)),
            pl.BlockSpec((1, 128), lambda i: (0, 0)),
            pl.BlockSpec((B, 1), lambda i: (0, 0)),
        ],
        out_shape=[
            jax.ShapeDtypeStruct((1, 128), jnp.float32),
            jax.ShapeDtypeStruct((1, 128), jnp.float32),
            jax.ShapeDtypeStruct((B, 1), jnp.float32),
        ],
    )(a2p8, cs8, qa8, qb8, bl2at, bl2bt, G128, theta,
      Wfc, bfc.reshape(1, F), Wn1, bn1.reshape(1, H), Wn2, bn2.reshape(1, H),
      Wn3, bn3.reshape(1, H), Wn4, bn4.reshape(1, H), Wno,
      bno.reshape(1, 2 * TD))
    return lp


def kernel(theta, x, edge_index, Wl1, bl1, Wr1, Wl2, bl2, Wr2, Wfc, bfc,
           Wn1, bn1, Wn2, bn2, Wn3, bn3, Wn4, bn4, Wno, bno):
    src = edge_index[0].astype(jnp.int32)
    dst = edge_index[1].astype(jnp.int32)
    pad = EP - E
    srcp = jnp.concatenate([src, jnp.zeros((pad,), jnp.int32)])
    dstp = jnp.concatenate([dst, jnp.full((pad,), N, jnp.int32)])
    zagg = jnp.zeros((NSLICE, F), jnp.float32)
    zcnt = jnp.zeros((NSLICE,), jnp.float32)

    I8 = jnp.eye(8, dtype=jnp.float32)
    KWl1 = jnp.kron(I8, Wl1.T)
    KWr1 = jnp.kron(I8, Wr1.T)
    KW2a = jnp.kron(I8, Wl2[:F].T)
    KW2b = jnp.kron(I8, Wl2[F:].T)
    KWra = jnp.kron(I8, Wr2[:F].T)
    KWrb = jnp.kron(I8, Wr2[F:].T)
    bl1t = jnp.tile(bl1, 8).reshape(1, 512)
    bl2at = jnp.tile(bl2[:F], 8).reshape(1, 128)
    bl2bt = jnp.tile(bl2[F:], 8).reshape(1, 128)
    G512 = jnp.kron(I8, jnp.ones((4 * F, 4 * F), jnp.float32))
    G128 = jnp.kron(I8, jnp.ones((F, F), jnp.float32))

    xflat = jnp.concatenate([x.reshape(N * F),
                             jnp.zeros(((NP - N) * F,), jnp.float32)])
    aggp, cntp = _sc_pass1(xflat.reshape(NP, F), srcp, dstp, zagg, zcnt)
    aggp8 = aggp.reshape(NC, NP8, 128)
    cnt8 = jnp.broadcast_to(cntp[:, :, None], (NC, NP, F)).reshape(NC, NP8, 128)
    xp = xflat.reshape(NP8, 128)

    p1a8, p1b8, qa8, qb8, cs8 = _dense1(aggp8, cnt8, xp, KWl1, bl1t, KWr1,
                                        KW2a, KW2b, KWra, KWrb, G512)
    agg2p = _sc_pass2(p1a8.reshape(NP, F), p1b8.reshape(NP, F),
                      srcp, dstp, zagg)
    lp = _dense2(agg2p.reshape(NC, NP8, 128), cs8, qa8, qb8,
                 bl2at, bl2bt, G128, theta, Wfc, bfc, Wn1, bn1, Wn2, bn2,
                 Wn3, bn3, Wn4, bn4, Wno, bno)
    return lp.reshape(B)
